# Initial kernel scaffold; baseline (speedup 1.0000x reference)
import jax
import jax.numpy as jnp
from jax.experimental import pallas as pl


def kernel(indices, minhash_table, hashed_weight):
    raise NotImplementedError("write your pallas kernel here")


# speedup vs baseline: 317.0183x; 317.0183x over previous
import functools

import jax
import jax.numpy as jnp
from jax import lax
from jax.experimental import pallas as pl
from jax.experimental.pallas import tpu as pltpu
from jax.experimental.pallas import tpu_sc as plsc

VOCAB = 100000
EMBED_DIM = 64
BATCH = 4096
HIST = 50
LSH_WEIGHT_SIZE = VOCAB * EMBED_DIM

NUM_CORES = 2
NUM_SUBCORES = 16
NUM_WORKERS = NUM_CORES * NUM_SUBCORES
BAGS_PER_WORKER = BATCH // NUM_WORKERS
CHUNK_BAGS = 8
NUM_CHUNKS = BAGS_PER_WORKER // CHUNK_BAGS
CHUNK_ROWS = CHUNK_BAGS * HIST
CHUNK_VALS = CHUNK_ROWS * EMBED_DIM
LANES = 16
VPR = EMBED_DIM // LANES


def _sc_body(idx_hbm, mh_hbm, w_hbm, out_hbm, idx_v, rows_v, flat_v, vals_v,
             out_v, sem_rows, sem_vals):
    wid = lax.axis_index("s") * NUM_CORES + lax.axis_index("c")
    base_bag = wid * BAGS_PER_WORKER
    pltpu.sync_copy(idx_hbm.at[pl.ds(base_bag * HIST, BAGS_PER_WORKER * HIST)],
                    idx_v)

    def chunk_body(c, _):
        pltpu.async_copy(
            mh_hbm.at[idx_v.at[pl.ds(c * CHUNK_ROWS, CHUNK_ROWS)]],
            rows_v, sem_rows).wait()
        def repack_body(r, _):
            for d in range(VPR):
                flat_v[pl.ds(r * EMBED_DIM + d * LANES, LANES)] = (
                    rows_v[r, pl.ds(d * LANES, LANES)])
            return 0

        lax.fori_loop(0, CHUNK_ROWS, repack_body, 0, unroll=4)

        pltpu.async_copy(w_hbm.at[flat_v], vals_v, sem_vals).wait()

        def bag_body(i, _):
            vbase = i * (HIST * EMBED_DIM)
            obase = (c * CHUNK_BAGS + i) * EMBED_DIM
            for d in range(VPR):
                acc = vals_v[pl.ds(vbase + d * LANES, LANES)]
                for h in range(1, HIST):
                    acc = acc + vals_v[pl.ds(vbase + h * EMBED_DIM + d * LANES,
                                             LANES)]
                out_v[pl.ds(obase + d * LANES, LANES)] = acc
            return 0

        lax.fori_loop(0, CHUNK_BAGS, bag_body, 0)
        return 0

    lax.fori_loop(0, NUM_CHUNKS, chunk_body, 0)

    pltpu.sync_copy(
        out_v,
        out_hbm.at[pl.ds(base_bag * EMBED_DIM, BAGS_PER_WORKER * EMBED_DIM)])


@jax.jit
def kernel(indices, minhash_table, hashed_weight):
    mesh = plsc.VectorSubcoreMesh(core_axis_name="c", subcore_axis_name="s",
                                  num_cores=NUM_CORES,
                                  num_subcores=NUM_SUBCORES)
    run = pl.kernel(
        _sc_body,
        out_type=jax.ShapeDtypeStruct((BATCH * EMBED_DIM,), jnp.float32),
        mesh=mesh,
        compiler_params=pltpu.CompilerParams(use_tc_tiling_on_sc=False),
        scratch_types=[
            pltpu.VMEM((BAGS_PER_WORKER * HIST,), jnp.int32),
            pltpu.VMEM((CHUNK_ROWS, EMBED_DIM), jnp.int32),
            pltpu.VMEM((CHUNK_VALS,), jnp.int32),
            pltpu.VMEM((CHUNK_VALS,), jnp.float32),
            pltpu.VMEM((BAGS_PER_WORKER * EMBED_DIM,), jnp.float32),
            pltpu.SemaphoreType.DMA,
            pltpu.SemaphoreType.DMA,
        ],
    )
    out = run(indices.reshape(-1), minhash_table, hashed_weight)
    return out.reshape(BATCH, EMBED_DIM)


# speedup vs baseline: 334.7946x; 1.0561x over previous
import jax
import jax.numpy as jnp
from jax import lax
from jax.experimental import pallas as pl
from jax.experimental.pallas import tpu as pltpu
from jax.experimental.pallas import tpu_sc as plsc

VOCAB = 100000
EMBED_DIM = 64
BATCH = 4096
HIST = 50
LSH_WEIGHT_SIZE = VOCAB * EMBED_DIM

NUM_CORES = 2
NUM_SUBCORES = 16
NUM_WORKERS = NUM_CORES * NUM_SUBCORES
BAGS_PER_WORKER = BATCH // NUM_WORKERS
CHUNK_BAGS = 4
NUM_CHUNKS = BAGS_PER_WORKER // CHUNK_BAGS
CHUNK_ROWS = CHUNK_BAGS * HIST
CHUNK_VALS = CHUNK_ROWS * EMBED_DIM
LANES = 16
VPR = EMBED_DIM // LANES


def _sc_body(idx_hbm, mh_hbm, w_hbm, out_hbm, idx_v,
             rows0, rows1, flat0, flat1, vals0, vals1, out_v,
             sem_rows, sem_vals):
    wid = lax.axis_index("s") * NUM_CORES + lax.axis_index("c")
    base_bag = wid * BAGS_PER_WORKER
    rows = (rows0, rows1)
    flat = (flat0, flat1)
    vals = (vals0, vals1)

    pltpu.sync_copy(idx_hbm.at[pl.ds(base_bag * HIST, BAGS_PER_WORKER * HIST)],
                    idx_v)

    def start_g1(c, buf):
        off = (c % NUM_CHUNKS) * CHUNK_ROWS
        pltpu.make_async_copy(
            mh_hbm.at[idx_v.at[pl.ds(off, CHUNK_ROWS)]], buf, sem_rows).start()

    def wait_g1(buf):
        pltpu.make_async_copy(
            mh_hbm.at[idx_v.at[pl.ds(0, CHUNK_ROWS)]], buf, sem_rows).wait()

    def repack(rbuf, fbuf):
        def body(r, _):
            for d in range(VPR):
                fbuf[pl.ds(r * EMBED_DIM + d * LANES, LANES)] = (
                    rbuf[r, pl.ds(d * LANES, LANES)])
            return 0
        lax.fori_loop(0, CHUNK_ROWS, body, 0, unroll=4)

    def reduce_chunk(c, vbuf):
        def bag_body(i, _):
            vbase = i * (HIST * EMBED_DIM)
            obase = (c * CHUNK_BAGS + i) * EMBED_DIM
            for d in range(VPR):
                acc = vbuf[pl.ds(vbase + d * LANES, LANES)]
                for h in range(1, HIST):
                    acc = acc + vbuf[pl.ds(vbase + h * EMBED_DIM + d * LANES,
                                           LANES)]
                out_v[pl.ds(obase + d * LANES, LANES)] = acc
            return 0
        lax.fori_loop(0, CHUNK_BAGS, bag_body, 0)

    start_g1(0, rows[0])

    def superstep(s, _):
        for p in range(2):
            c = s * 2 + p
            q = 1 - p
            wait_g1(rows[p])
            repack(rows[p], flat[p])
            pltpu.make_async_copy(w_hbm.at[flat[p]], vals[p], sem_vals).start()
            start_g1(c + 1, rows[q])

            @pl.when(c > 0)
            def _():
                reduce_chunk(c - 1, vals[q])

            pltpu.make_async_copy(w_hbm.at[flat[p]], vals[p], sem_vals).wait()
        return 0

    lax.fori_loop(0, NUM_CHUNKS // 2, superstep, 0)

    reduce_chunk(NUM_CHUNKS - 1, vals[1])
    wait_g1(rows[0])

    pltpu.sync_copy(
        out_v,
        out_hbm.at[pl.ds(base_bag * EMBED_DIM, BAGS_PER_WORKER * EMBED_DIM)])


@jax.jit
def kernel(indices, minhash_table, hashed_weight):
    mesh = plsc.VectorSubcoreMesh(core_axis_name="c", subcore_axis_name="s",
                                  num_cores=NUM_CORES,
                                  num_subcores=NUM_SUBCORES)
    run = pl.kernel(
        _sc_body,
        out_type=jax.ShapeDtypeStruct((BATCH * EMBED_DIM,), jnp.float32),
        mesh=mesh,
        compiler_params=pltpu.CompilerParams(use_tc_tiling_on_sc=False),
        scratch_types=[
            pltpu.VMEM((BAGS_PER_WORKER * HIST,), jnp.int32),
            pltpu.VMEM((CHUNK_ROWS, EMBED_DIM), jnp.int32),
            pltpu.VMEM((CHUNK_ROWS, EMBED_DIM), jnp.int32),
            pltpu.VMEM((CHUNK_VALS,), jnp.int32),
            pltpu.VMEM((CHUNK_VALS,), jnp.int32),
            pltpu.VMEM((CHUNK_VALS,), jnp.float32),
            pltpu.VMEM((CHUNK_VALS,), jnp.float32),
            pltpu.VMEM((BAGS_PER_WORKER * EMBED_DIM,), jnp.float32),
            pltpu.SemaphoreType.DMA,
            pltpu.SemaphoreType.DMA,
        ],
    )
    out = run(indices.reshape(-1), minhash_table, hashed_weight)
    return out.reshape(BATCH, EMBED_DIM)


# speedup vs baseline: 596.6155x; 1.7820x over previous
import jax
import jax.numpy as jnp
from jax import lax
from jax.experimental import pallas as pl
from jax.experimental.pallas import tpu as pltpu
from jax.experimental.pallas import tpu_sc as plsc

VOCAB = 100000
EMBED_DIM = 64
BATCH = 4096
HIST = 50
LSH_WEIGHT_SIZE = VOCAB * EMBED_DIM

NUM_CORES = 2
NUM_SUBCORES = 16
NUM_WORKERS = NUM_CORES * NUM_SUBCORES
LANES = 16
VPR = EMBED_DIM // LANES

P1_PER_TILE = VOCAB * EMBED_DIM // NUM_WORKERS
P1_CHUNK = 20000
P1_CHUNKS = P1_PER_TILE // P1_CHUNK

BAGS_PER_WORKER = BATCH // NUM_WORKERS
CHUNK_BAGS = 8
NUM_CHUNKS = BAGS_PER_WORKER // CHUNK_BAGS
CHUNK_ROWS = CHUNK_BAGS * HIST


def _p1_body(mh_hbm, w_hbm, vt_hbm, midx0, midx1, wval0, wval1,
             sem_m, sem_g, sem_s):
    wid = lax.axis_index("s") * NUM_CORES + lax.axis_index("c")
    base = wid * P1_PER_TILE
    midx = (midx0, midx1)
    wval = (wval0, wval1)

    def start_mload(c, buf):
        off = base + (c % P1_CHUNKS) * P1_CHUNK
        pltpu.make_async_copy(mh_hbm.at[pl.ds(off, P1_CHUNK)], buf,
                              sem_m).start()

    def start_store(c, buf):
        off = base + c * P1_CHUNK
        pltpu.make_async_copy(buf, vt_hbm.at[pl.ds(off, P1_CHUNK)],
                              sem_s).start()

    start_mload(0, midx[0])

    def superstep(s, _):
        for p in range(2):
            c = s * 2 + p
            q = 1 - p
            pltpu.make_async_copy(mh_hbm.at[pl.ds(base, P1_CHUNK)], midx[p],
                                  sem_m).wait()
            start_mload(c + 1, midx[q])
            @pl.when(c >= 2)
            def _():
                pltpu.make_async_copy(wval[p],
                                      vt_hbm.at[pl.ds(base, P1_CHUNK)],
                                      sem_s).wait()
            pltpu.async_copy(w_hbm.at[midx[p]], wval[p], sem_g).wait()
            start_store(c, wval[p])
        return 0

    lax.fori_loop(0, P1_CHUNKS // 2, superstep, 0)

    pltpu.make_async_copy(mh_hbm.at[pl.ds(base, P1_CHUNK)], midx[0],
                          sem_m).wait()
    for p in range(2):
        pltpu.make_async_copy(wval[p], vt_hbm.at[pl.ds(base, P1_CHUNK)],
                              sem_s).wait()


def _p2_body(idx_hbm, vt_hbm, out_hbm, idx_v, vals0, vals1, out_v, sem_r):
    wid = lax.axis_index("s") * NUM_CORES + lax.axis_index("c")
    base_bag = wid * BAGS_PER_WORKER
    vals = (vals0, vals1)

    pltpu.sync_copy(idx_hbm.at[pl.ds(base_bag * HIST, BAGS_PER_WORKER * HIST)],
                    idx_v)

    def start_gather(c, buf):
        off = (c % NUM_CHUNKS) * CHUNK_ROWS
        pltpu.make_async_copy(
            vt_hbm.at[idx_v.at[pl.ds(off, CHUNK_ROWS)]], buf, sem_r).start()

    def wait_gather(buf):
        pltpu.make_async_copy(
            vt_hbm.at[idx_v.at[pl.ds(0, CHUNK_ROWS)]], buf, sem_r).wait()

    def reduce_chunk(c, vbuf):
        def bag_body(i, _):
            rbase = i * HIST
            obase = (c * CHUNK_BAGS + i) * EMBED_DIM
            for d in range(VPR):
                acc = vbuf[rbase, pl.ds(d * LANES, LANES)]
                for h in range(1, HIST):
                    acc = acc + vbuf[rbase + h, pl.ds(d * LANES, LANES)]
                out_v[pl.ds(obase + d * LANES, LANES)] = acc
            return 0
        lax.fori_loop(0, CHUNK_BAGS, bag_body, 0)

    start_gather(0, vals[0])

    def superstep(s, _):
        for p in range(2):
            c = s * 2 + p
            q = 1 - p
            wait_gather(vals[p])
            start_gather(c + 1, vals[q])
            reduce_chunk(c, vals[p])
        return 0

    lax.fori_loop(0, NUM_CHUNKS // 2, superstep, 0)

    wait_gather(vals[0])

    pltpu.sync_copy(
        out_v,
        out_hbm.at[pl.ds(base_bag * EMBED_DIM, BAGS_PER_WORKER * EMBED_DIM)])


@jax.jit
def kernel(indices, minhash_table, hashed_weight):
    mesh = plsc.VectorSubcoreMesh(core_axis_name="c", subcore_axis_name="s",
                                  num_cores=NUM_CORES,
                                  num_subcores=NUM_SUBCORES)
    params = pltpu.CompilerParams(use_tc_tiling_on_sc=False)

    build_vt = pl.kernel(
        _p1_body,
        out_type=jax.ShapeDtypeStruct((VOCAB * EMBED_DIM,), jnp.float32),
        mesh=mesh,
        compiler_params=params,
        scratch_types=[
            pltpu.VMEM((P1_CHUNK,), jnp.int32),
            pltpu.VMEM((P1_CHUNK,), jnp.int32),
            pltpu.VMEM((P1_CHUNK,), jnp.float32),
            pltpu.VMEM((P1_CHUNK,), jnp.float32),
            pltpu.SemaphoreType.DMA,
            pltpu.SemaphoreType.DMA,
            pltpu.SemaphoreType.DMA,
        ],
    )
    bag_reduce = pl.kernel(
        _p2_body,
        out_type=jax.ShapeDtypeStruct((BATCH * EMBED_DIM,), jnp.float32),
        mesh=mesh,
        compiler_params=params,
        scratch_types=[
            pltpu.VMEM((BAGS_PER_WORKER * HIST,), jnp.int32),
            pltpu.VMEM((CHUNK_ROWS, EMBED_DIM), jnp.float32),
            pltpu.VMEM((CHUNK_ROWS, EMBED_DIM), jnp.float32),
            pltpu.VMEM((BAGS_PER_WORKER * EMBED_DIM,), jnp.float32),
            pltpu.SemaphoreType.DMA,
        ],
    )

    vt = build_vt(minhash_table.reshape(-1), hashed_weight)
    out = bag_reduce(indices.reshape(-1), vt.reshape(VOCAB, EMBED_DIM))
    return out.reshape(BATCH, EMBED_DIM)
